# baseline (speedup 1.0000x reference)
import jax
import jax.numpy as jnp
from jax.experimental import pallas as pl
from jax.experimental.pallas import tpu as pltpu

TN = 2048


def _fused_kernel(x_ref, bc1_ref, b1_ref, bc2_ref, b2_ref,
                  w3_ref, b3_ref, w4_ref, b4_ref, out_ref):
    x = x_ref[...]
    bc1 = bc1_ref[...]
    b1 = b1_ref[...]

    def dot1(b, s):
        return jnp.dot(b, s, preferred_element_type=jnp.float32)

    def pool4(z, m):
        return jnp.maximum(jnp.maximum(z[:m], z[m:2 * m]),
                           jnp.maximum(z[2 * m:3 * m], z[3 * m:]))

    zpad = jnp.zeros((8, TN), jnp.bfloat16)
    p1_rows = []
    for t in range(12):
        z = dot1(bc1, x[2 * t:2 * t + 6].reshape(192, TN))
        r = jnp.maximum(pool4(z, 120) + b1, 0.0).astype(jnp.bfloat16)
        p1_rows.append(jnp.concatenate([r, zpad], axis=0))
    p1 = jnp.stack(p1_rows, axis=0)

    bc2 = bc2_ref[...]
    b2 = b2_ref[...]

    p2_rows = []
    for t in range(4):
        z = dot1(bc2, p1[2 * t:2 * t + 6].reshape(768, TN))
        m = pool4(z, 80)
        p2_rows.append(jnp.maximum(m + b2, 0.0).astype(jnp.bfloat16))
    p2 = jnp.stack(p2_rows, axis=0)

    act = p2.reshape(320, TN)
    h1 = jnp.dot(w3_ref[...], act, preferred_element_type=jnp.float32)
    h1 = jnp.maximum(h1 + b3_ref[...], 0.0)

    logits = jnp.dot(w4_ref[...], h1,
                     preferred_element_type=jnp.float32) + b4_ref[...]
    m = jnp.max(logits, axis=0, keepdims=True)
    s = logits - m
    lse = jnp.log(jnp.sum(jnp.exp(s), axis=0, keepdims=True))
    out_ref[...] = s - lse


def kernel(x, conv1_w, conv1_b, conv2_w, conv2_b, fc1_w, fc1_b, fc2_w, fc2_b):
    N, C, H, W = x.shape
    assert (C, H, W) == (1, 28, 28), "Net requires 1x28x28 inputs"
    npad = ((N + TN - 1) // TN) * TN

    xt = x.reshape(N, 28, 28).transpose(1, 2, 0).astype(jnp.bfloat16)
    x2 = jnp.pad(xt, ((0, 0), (0, 4), (0, npad - N)))

    w1r = conv1_w.reshape(10, 5, 5)
    par = jnp.arange(2)[:, None, None]

    hs = (jnp.arange(6)[None, :, None]
          == jnp.arange(5)[None, None, :] + par).astype(jnp.float32)

    def esel(n_out, n_w):
        return (jnp.arange(n_w)[None, None, :, None]
                == 2 * jnp.arange(n_out)[None, :, None, None]
                + jnp.arange(5)[None, None, None, :]
                + par[..., None]).astype(jnp.float32)

    bc1 = jnp.einsum("prh,qxwk,chk->pqcxrw", hs, esel(12, 32), w1r)
    bc1 = bc1.reshape(480, 192).astype(jnp.bfloat16)
    b1r = jnp.broadcast_to(conv1_b[:, None], (10, 12)).reshape(120, 1)

    bc2 = jnp.einsum("prh,qxwk,oihk->pqoxriw", hs, esel(4, 12), conv2_w)
    bc2 = jnp.pad(bc2.reshape(320, 6, 120), ((0, 0), (0, 0), (0, 8)))
    bc2 = bc2.reshape(320, 768).astype(jnp.bfloat16)
    b2r = jnp.broadcast_to(conv2_b[:, None], (20, 4)).reshape(80, 1)

    w3p = fc1_w.reshape(300, 20, 4, 4).transpose(0, 2, 1, 3).reshape(300, 320)
    w3p = w3p.astype(jnp.bfloat16)
    b3c = fc1_b.reshape(300, 1)
    b4c = fc2_b.reshape(10, 1)

    out = pl.pallas_call(
        _fused_kernel,
        out_shape=jax.ShapeDtypeStruct((10, npad), jnp.float32),
        grid=(npad // TN,),
        in_specs=[
            pl.BlockSpec((28, 32, TN), lambda b: (0, 0, b)),
            pl.BlockSpec((480, 192), lambda b: (0, 0)),
            pl.BlockSpec((120, 1), lambda b: (0, 0)),
            pl.BlockSpec((320, 768), lambda b: (0, 0)),
            pl.BlockSpec((80, 1), lambda b: (0, 0)),
            pl.BlockSpec((300, 320), lambda b: (0, 0)),
            pl.BlockSpec((300, 1), lambda b: (0, 0)),
            pl.BlockSpec((10, 300), lambda b: (0, 0)),
            pl.BlockSpec((10, 1), lambda b: (0, 0)),
        ],
        out_specs=pl.BlockSpec((10, TN), lambda b: (0, b)),
        compiler_params=pltpu.CompilerParams(
            dimension_semantics=("parallel",),
            vmem_limit_bytes=56 * 1024 * 1024,
        ),
    )(x2, bc1, b1r, bc2, b2r, w3p, b3c, fc2_w, b4c)

    return out[:, :N].T


# speedup vs baseline: 1.0123x; 1.0123x over previous
import jax
import jax.numpy as jnp
from jax.experimental import pallas as pl
from jax.experimental.pallas import tpu as pltpu

TN = 2048


def _fused_kernel(x_ref, bc1_ref, b1_ref, bc2_ref, b2_ref,
                  w3_ref, b3_ref, w4_ref, b4_ref, out_ref):
    x = x_ref[...]
    bc1 = bc1_ref[...]
    b1 = b1_ref[...]

    def dot1(b, s):
        return jnp.dot(b, s, preferred_element_type=jnp.float32)

    def pool4(z, m):
        return jnp.maximum(jnp.maximum(z[:m], z[m:2 * m]),
                           jnp.maximum(z[2 * m:3 * m], z[3 * m:]))

    zpad = jnp.zeros((8, TN), jnp.bfloat16)
    p1_rows = []
    for t in range(12):
        z = dot1(bc1, x[2 * t:2 * t + 6].reshape(192, TN))
        r = jnp.maximum(pool4(z, 120) + b1, 0.0).astype(jnp.bfloat16)
        p1_rows.append(jnp.concatenate([r, zpad], axis=0))
    p1 = jnp.stack(p1_rows, axis=0)

    bc2 = bc2_ref[...]
    b2 = b2_ref[...]

    p2_rows = []
    for t in range(4):
        z = dot1(bc2, p1[2 * t:2 * t + 6].reshape(768, TN))
        m = pool4(z, 80)
        p2_rows.append(jnp.maximum(m + b2, 0.0).astype(jnp.bfloat16))
    p2 = jnp.stack(p2_rows, axis=0)

    act = p2.reshape(320, TN)
    h1 = jnp.dot(w3_ref[...], act, preferred_element_type=jnp.float32)
    h1 = jnp.maximum(h1 + b3_ref[...], 0.0)

    logits = jnp.dot(w4_ref[...], h1,
                     preferred_element_type=jnp.float32) + b4_ref[...]
    m = jnp.max(logits, axis=0, keepdims=True)
    s = logits - m
    lse = jnp.log(jnp.sum(jnp.exp(s), axis=0, keepdims=True))
    out_ref[...] = s - lse


def kernel(x, conv1_w, conv1_b, conv2_w, conv2_b, fc1_w, fc1_b, fc2_w, fc2_b):
    N, C, H, W = x.shape
    assert (C, H, W) == (1, 28, 28), "Net requires 1x28x28 inputs"
    npad = ((N + TN - 1) // TN) * TN

    xt = x.reshape(N, 28, 28).transpose(1, 2, 0).astype(jnp.bfloat16)
    x2 = jnp.pad(xt, ((0, 0), (0, 4), (0, npad - N)))

    w1r = conv1_w.reshape(10, 5, 5)
    parities = ((0, 0), (0, 1), (1, 0), (1, 1))

    def qband(n_out, n_w, po, pw, wt, spec):
        h = (jnp.arange(6)[:, None]
             == jnp.arange(5)[None, :] + po).astype(jnp.float32)
        e = (jnp.arange(n_w)[None, :, None]
             == 2 * jnp.arange(n_out)[:, None, None]
             + jnp.arange(5)[None, None, :] + pw).astype(jnp.float32)
        return jnp.einsum(spec, h, e, wt)

    bc1 = jnp.concatenate(
        [qband(12, 32, po, pw, w1r, "rh,xwk,chk->cxrw").reshape(120, 192)
         for po, pw in parities]).astype(jnp.bfloat16)
    b1r = jnp.broadcast_to(conv1_b[:, None], (10, 12)).reshape(120, 1)

    bc2 = jnp.concatenate(
        [jnp.pad(qband(4, 12, po, pw, conv2_w,
                       "rh,xwk,oihk->oxriw").reshape(80, 6, 120),
                 ((0, 0), (0, 0), (0, 8))).reshape(80, 768)
         for po, pw in parities]).astype(jnp.bfloat16)
    b2r = jnp.broadcast_to(conv2_b[:, None], (20, 4)).reshape(80, 1)

    w3p = fc1_w.reshape(300, 20, 4, 4).transpose(0, 2, 1, 3).reshape(300, 320)
    w3p = w3p.astype(jnp.bfloat16)
    b3c = fc1_b.reshape(300, 1)
    b4c = fc2_b.reshape(10, 1)

    out = pl.pallas_call(
        _fused_kernel,
        out_shape=jax.ShapeDtypeStruct((10, npad), jnp.float32),
        grid=(npad // TN,),
        in_specs=[
            pl.BlockSpec((28, 32, TN), lambda b: (0, 0, b)),
            pl.BlockSpec((480, 192), lambda b: (0, 0)),
            pl.BlockSpec((120, 1), lambda b: (0, 0)),
            pl.BlockSpec((320, 768), lambda b: (0, 0)),
            pl.BlockSpec((80, 1), lambda b: (0, 0)),
            pl.BlockSpec((300, 320), lambda b: (0, 0)),
            pl.BlockSpec((300, 1), lambda b: (0, 0)),
            pl.BlockSpec((10, 300), lambda b: (0, 0)),
            pl.BlockSpec((10, 1), lambda b: (0, 0)),
        ],
        out_specs=pl.BlockSpec((10, TN), lambda b: (0, b)),
        compiler_params=pltpu.CompilerParams(
            dimension_semantics=("parallel",),
            vmem_limit_bytes=56 * 1024 * 1024,
        ),
    )(x2, bc1, b1r, bc2, b2r, w3p, b3c, fc2_w, b4c)

    return out[:, :N].T
